# baseline (speedup 1.0000x reference)
import functools

import jax
import jax.numpy as jnp
from jax import lax
from jax.experimental import pallas as pl
from jax.experimental.pallas import tpu as pltpu
from jax.experimental.pallas import tpu_sc as plsc

_NCLS = 21
_NBINS = _NCLS * _NCLS
_BINS_PAD = 512
_NROWS = 32 * 512
_NW = 32
_ROWS_W = _NROWS // _NW
_CROWS = 16
_CHUNK = _CROWS * 512
_NCHUNK = _ROWS_W // _CROWS
_UNROLL = 6


def _sc_hist_kernel(pr_hbm, gt_hbm, out_hbm, pr_v, gt_v, hist_v, red_v,
                    pr_s, gt_s, sa, sb):
    cid = lax.axis_index("c")
    sid = lax.axis_index("s")
    wid = sid * 2 + cid
    row_base = wid * _ROWS_W

    zeros16 = jnp.zeros((16,), jnp.int32)
    ones16 = jnp.ones((16,), jnp.int32)
    lane_base = lax.iota(jnp.int32, 16) * _BINS_PAD

    @plsc.parallel_loop(0, (16 * _BINS_PAD) // 16, 1, unroll=8)
    def z_body(i):
        hist_v[pl.ds(i * 16, 16)] = zeros16

    def start_a(ci, b):
        r0 = row_base + ci * _CROWS
        pltpu.make_async_copy(
            pr_hbm.at[pl.ds(r0, _CROWS), :], pr_s.at[sid, b], sa.at[b, 0]).start()
        pltpu.make_async_copy(
            gt_hbm.at[pl.ds(r0, _CROWS), :], gt_s.at[sid, b], sa.at[b, 1]).start()

    def wait_a(b):
        pltpu.make_async_copy(
            pr_hbm.at[pl.ds(row_base, _CROWS), :], pr_s.at[sid, b],
            sa.at[b, 0]).wait()
        pltpu.make_async_copy(
            gt_hbm.at[pl.ds(row_base, _CROWS), :], gt_s.at[sid, b],
            sa.at[b, 1]).wait()

    def start_b(b):
        pltpu.make_async_copy(pr_s.at[sid, b], pr_v.at[b], sb.at[b, 0]).start()
        pltpu.make_async_copy(gt_s.at[sid, b], gt_v.at[b], sb.at[b, 1]).start()

    def wait_b(b):
        pltpu.make_async_copy(pr_s.at[sid, b], pr_v.at[b], sb.at[b, 0]).wait()
        pltpu.make_async_copy(gt_s.at[sid, b], gt_v.at[b], sb.at[b, 1]).wait()

    def compute(b):
        @plsc.parallel_loop(0, _CHUNK // 16, 1, unroll=_UNROLL)
        def inner(i):
            r = lax.shift_right_logical(i, 5)
            c = lax.shift_left(lax.bitwise_and(i, 31), 4)
            p = pr_v[b, r, pl.ds(c, 16)]
            g = gt_v[b, r, pl.ds(c, 16)]
            idx = lane_base + g * _NCLS + p
            plsc.addupdate_scatter(hist_v, [idx], ones16)

    start_a(0, 0)
    wait_a(0)
    start_b(0)
    start_a(1, 1)

    def outer(i, c):
        ci0 = i * 3
        for k in range(3):
            ci = ci0 + k
            wait_b(k)

            nci_a = ci + 2

            @pl.when(nci_a < _NCHUNK)
            def _():
                start_a(nci_a, (k + 2) % 3)

            @pl.when(ci + 1 < _NCHUNK)
            def _():
                wait_a((k + 1) % 3)
                start_b((k + 1) % 3)

            compute(k)
        return c

    lax.fori_loop(0, _NCHUNK // 3, outer, 0)
    wait_b((_NCHUNK - 2) % 3)
    wait_a((_NCHUNK - 1) % 3)
    start_b((_NCHUNK - 1) % 3)
    compute((_NCHUNK - 2) % 3)
    wait_b((_NCHUNK - 1) % 3)
    compute((_NCHUNK - 1) % 3)

    @plsc.parallel_loop(0, _BINS_PAD // 16, 1, unroll=2)
    def red_body(j):
        acc = hist_v[pl.ds(j * 16, 16)]
        for l in range(1, 16):
            acc = acc + hist_v[pl.ds(l * _BINS_PAD + j * 16, 16)]
        red_v[pl.ds(j * 16, 16)] = acc

    pltpu.sync_copy(red_v, out_hbm.at[wid])


def _metric_body(h_ref, o_ref):
    x = h_ref[...]
    conf = jnp.sum(x, axis=0, keepdims=True).astype(jnp.float32)

    bb = lax.broadcasted_iota(jnp.int32, (32, _BINS_PAD), 1)
    cc = lax.broadcasted_iota(jnp.int32, (32, _BINS_PAD), 0)
    gg = bb // _NCLS
    pp = bb - gg * _NCLS
    vmask = bb < _NBINS

    confb = jnp.broadcast_to(conf, (32, _BINS_PAD))
    zero = jnp.zeros((32, _BINS_PAD), jnp.float32)
    row = jnp.sum(jnp.where(vmask & (gg == cc), confb, zero), axis=1,
                  keepdims=True)
    col = jnp.sum(jnp.where(vmask & (pp == cc), confb, zero), axis=1,
                  keepdims=True)
    tp = jnp.sum(jnp.where(vmask & (gg == cc) & (pp == cc), confb, zero),
                 axis=1, keepdims=True)
    total = jnp.sum(conf)

    fp = col - tp
    fn = row - tp
    tn = total - tp - fn - fp
    pa = (tp + tn) / total
    cls_valid = row > 0
    pa_sum = jnp.sum(jnp.where(cls_valid, pa, jnp.zeros_like(pa)))
    n_valid = jnp.sum(cls_valid.astype(jnp.float32))
    o_ref[0, 0] = pa_sum / n_valid


@jax.jit
def kernel(y_pr, y_gt):
    pr = y_pr.reshape(_NROWS, 512).astype(jnp.int32)
    gt = y_gt.reshape(_NROWS, 512).astype(jnp.int32)

    mesh = plsc.VectorSubcoreMesh(core_axis_name="c", subcore_axis_name="s")
    hist = functools.partial(
        pl.kernel,
        mesh=mesh,
        compiler_params=pltpu.CompilerParams(needs_layout_passes=False),
        out_type=jax.ShapeDtypeStruct((_NW, _BINS_PAD), jnp.int32),
        scratch_types=[
            pltpu.VMEM((3, _CROWS, 512), jnp.int32),
            pltpu.VMEM((3, _CROWS, 512), jnp.int32),
            pltpu.VMEM((16 * _BINS_PAD,), jnp.int32),
            pltpu.VMEM((_BINS_PAD,), jnp.int32),
            pltpu.VMEM_SHARED((16, 3, _CROWS, 512), jnp.int32),
            pltpu.VMEM_SHARED((16, 3, _CROWS, 512), jnp.int32),
            pltpu.SemaphoreType.DMA((3, 2)),
            pltpu.SemaphoreType.DMA((3, 2)),
        ],
    )(_sc_hist_kernel)(pr, gt)

    out = pl.pallas_call(
        _metric_body,
        out_shape=jax.ShapeDtypeStruct((1, 1), jnp.float32),
        out_specs=pl.BlockSpec(memory_space=pltpu.SMEM),
    )(hist)
    return out[0, 0]


# speedup vs baseline: 1.1403x; 1.1403x over previous
import functools

import jax
import jax.numpy as jnp
from jax import lax
from jax.experimental import pallas as pl
from jax.experimental.pallas import tpu as pltpu
from jax.experimental.pallas import tpu_sc as plsc

_NCLS = 21
_NBINS = _NCLS * _NCLS
_BINS_PAD = 512
_NROWS = 32 * 512
_NW = 32
_ROWS_W = _NROWS // _NW
_CROWS = 32
_CHUNK = _CROWS * 512
_NCHUNK = _ROWS_W // _CROWS
_UNROLL = 6


def _sc_hist_kernel(pr_hbm, gt_hbm, out_hbm, pr_v, gt_v, hist_a, hist_b,
                    red_v, sp0, sp1, sg0, sg1):
    wid = lax.axis_index("s") * 2 + lax.axis_index("c")
    row_base = wid * _ROWS_W

    zeros16 = jnp.zeros((16,), jnp.int32)
    ones16 = jnp.ones((16,), jnp.int32)
    lane_base = lax.iota(jnp.int32, 16) * _BINS_PAD

    @plsc.parallel_loop(0, (16 * _BINS_PAD) // 16, 1, unroll=8)
    def z_body(i):
        hist_a[pl.ds(i * 16, 16)] = zeros16
        hist_b[pl.ds(i * 16, 16)] = zeros16

    def start(ci, b):
        r0 = row_base + ci * _CROWS
        sp = (sp0, sp1)[b]
        sg = (sg0, sg1)[b]
        pltpu.make_async_copy(
            pr_hbm.at[pl.ds(r0, _CROWS), :], pr_v.at[b], sp).start()
        pltpu.make_async_copy(
            gt_hbm.at[pl.ds(r0, _CROWS), :], gt_v.at[b], sg).start()

    def wait(b):
        sp = (sp0, sp1)[b]
        sg = (sg0, sg1)[b]
        pltpu.make_async_copy(
            pr_hbm.at[pl.ds(row_base, _CROWS), :], pr_v.at[b], sp).wait()
        pltpu.make_async_copy(
            gt_hbm.at[pl.ds(row_base, _CROWS), :], gt_v.at[b], sg).wait()

    def compute(b):
        @plsc.parallel_loop(0, _CHUNK // 32, 1, unroll=_UNROLL)
        def inner(i):
            r = lax.shift_right_logical(i, 4)
            c = lax.shift_left(lax.bitwise_and(i, 15), 5)
            p0 = pr_v[b, r, pl.ds(c, 16)]
            g0 = gt_v[b, r, pl.ds(c, 16)]
            idx0 = lane_base + g0 * _NCLS + p0
            plsc.addupdate_scatter(hist_a, [idx0], ones16)
            p1 = pr_v[b, r, pl.ds(c + 16, 16)]
            g1 = gt_v[b, r, pl.ds(c + 16, 16)]
            idx1 = lane_base + g1 * _NCLS + p1
            plsc.addupdate_scatter(hist_b, [idx1], ones16)

    start(0, 0)

    def outer(i, c):
        ci = i * 2
        start(ci + 1, 1)
        wait(0)
        compute(0)

        @pl.when(i < _NCHUNK // 2 - 1)
        def _():
            start(ci + 2, 0)

        wait(1)
        compute(1)
        return c

    lax.fori_loop(0, _NCHUNK // 2, outer, 0)

    @plsc.parallel_loop(0, _BINS_PAD // 16, 1, unroll=2)
    def red_body(j):
        acc = hist_a[pl.ds(j * 16, 16)] + hist_b[pl.ds(j * 16, 16)]
        for l in range(1, 16):
            acc = acc + hist_a[pl.ds(l * _BINS_PAD + j * 16, 16)]
            acc = acc + hist_b[pl.ds(l * _BINS_PAD + j * 16, 16)]
        red_v[pl.ds(j * 16, 16)] = acc

    pltpu.sync_copy(red_v, out_hbm.at[wid])


def _metric_body(h_ref, o_ref):
    x = h_ref[...]
    conf = jnp.sum(x, axis=0, keepdims=True).astype(jnp.float32)

    bb = lax.broadcasted_iota(jnp.int32, (32, _BINS_PAD), 1)
    cc = lax.broadcasted_iota(jnp.int32, (32, _BINS_PAD), 0)
    gg = bb // _NCLS
    pp = bb - gg * _NCLS
    vmask = bb < _NBINS

    confb = jnp.broadcast_to(conf, (32, _BINS_PAD))
    zero = jnp.zeros((32, _BINS_PAD), jnp.float32)
    row = jnp.sum(jnp.where(vmask & (gg == cc), confb, zero), axis=1,
                  keepdims=True)
    col = jnp.sum(jnp.where(vmask & (pp == cc), confb, zero), axis=1,
                  keepdims=True)
    tp = jnp.sum(jnp.where(vmask & (gg == cc) & (pp == cc), confb, zero),
                 axis=1, keepdims=True)
    total = jnp.sum(conf)

    fp = col - tp
    fn = row - tp
    tn = total - tp - fn - fp
    pa = (tp + tn) / total
    cls_valid = row > 0
    pa_sum = jnp.sum(jnp.where(cls_valid, pa, jnp.zeros_like(pa)))
    n_valid = jnp.sum(cls_valid.astype(jnp.float32))
    o_ref[0, 0] = pa_sum / n_valid


@jax.jit
def kernel(y_pr, y_gt):
    pr = y_pr.reshape(_NROWS, 512).astype(jnp.int32)
    gt = y_gt.reshape(_NROWS, 512).astype(jnp.int32)

    mesh = plsc.VectorSubcoreMesh(core_axis_name="c", subcore_axis_name="s")
    hist = functools.partial(
        pl.kernel,
        mesh=mesh,
        compiler_params=pltpu.CompilerParams(
            needs_layout_passes=False, skip_device_barrier=True),
        out_type=jax.ShapeDtypeStruct((_NW, _BINS_PAD), jnp.int32),
        scratch_types=[
            pltpu.VMEM((2, _CROWS, 512), jnp.int32),
            pltpu.VMEM((2, _CROWS, 512), jnp.int32),
            pltpu.VMEM((16 * _BINS_PAD,), jnp.int32),
            pltpu.VMEM((16 * _BINS_PAD,), jnp.int32),
            pltpu.VMEM((_BINS_PAD,), jnp.int32),
            pltpu.SemaphoreType.DMA,
            pltpu.SemaphoreType.DMA,
            pltpu.SemaphoreType.DMA,
            pltpu.SemaphoreType.DMA,
        ],
    )(_sc_hist_kernel)(pr, gt)

    out = pl.pallas_call(
        _metric_body,
        out_shape=jax.ShapeDtypeStruct((1, 1), jnp.float32),
        out_specs=pl.BlockSpec(memory_space=pltpu.SMEM),
    )(hist)
    return out[0, 0]
